# baseline (speedup 1.0000x reference)
import functools

import jax
import jax.numpy as jnp
from jax import lax
from jax.experimental import pallas as pl
from jax.experimental.pallas import tpu as pltpu
from jax.experimental.pallas import tpu_sc as plsc

_N = 50000
_E = 1600000
_P = 25000
_NUM_LAYER = 6

_NC = 2
_NS = 16
_NW = _NC * _NS
_STRIPE = 3136
_NP = _NS * _STRIPE
_EPW = _E // _NW
_CH = 2000
_NCHUNK = _EPW // _CH


def _sc_spmm_body(w, cols_hbm, rc_hbm, val_hbm, out_hbm,
                  htab, acc, rcb, valb, bcast):
    cid = lax.axis_index("c")
    sid = lax.axis_index("s")
    wid = cid * _NS + sid
    ebase = wid * _EPW
    zeros16 = jnp.zeros((16,), jnp.float32)

    for c in range(w):
        @pl.when(sid == 0)
        def _():
            pltpu.sync_copy(cols_hbm.at[pl.ds(c * _NP, _NP)], bcast)
        plsc.subcore_barrier()
        pltpu.sync_copy(bcast, htab)

        def zbody(j, carry):
            acc[pl.ds(j * 16, 16)] = zeros16
            return carry
        lax.fori_loop(0, _NP // 16, zbody, 0)

        def chunk_body(ck, carry):
            base = ebase + ck * _CH
            pltpu.sync_copy(rc_hbm.at[pl.ds(base, _CH)], rcb)
            pltpu.sync_copy(val_hbm.at[pl.ds(base, _CH)], valb)

            def it_body(t, icarry):
                rc = rcb[pl.ds(t * 16, 16)]
                vv = valb[pl.ds(t * 16, 16)]
                colv = lax.bitwise_and(rc, jnp.int32(0xFFFF))
                rowv = lax.shift_right_logical(rc, 16)
                g = plsc.load_gather(htab, [colv])
                msg = vv * g
                cnt, _ = plsc.scan_count(rowv)
                plsc.addupdate_scatter(acc, [rowv], msg, mask=cnt == 1)
                plsc.addupdate_scatter(acc, [rowv], msg, mask=cnt == 2)

                @pl.when(jnp.max(cnt) > 2)
                def _():
                    for k in range(3, 17):
                        plsc.addupdate_scatter(acc, [rowv], msg,
                                               mask=cnt == k)
                return icarry
            lax.fori_loop(0, _CH // 16, it_body, 0)
            return carry
        lax.fori_loop(0, _NCHUNK, chunk_body, 0)

        pltpu.sync_copy(acc, out_hbm.at[pl.ds((c * _NW + wid) * _NP, _NP)])
        plsc.subcore_barrier()


@functools.cache
def _make_sc_spmm(w):
    mesh = plsc.VectorSubcoreMesh(core_axis_name="c", subcore_axis_name="s", num_cores=_NC, num_subcores=_NS)
    return pl.kernel(
        functools.partial(_sc_spmm_body, w),
        out_type=jax.ShapeDtypeStruct((w * _NW * _NP,), jnp.float32),
        mesh=mesh,
        scratch_types=[
            pltpu.VMEM((_NP,), jnp.float32),
            pltpu.VMEM((_NP,), jnp.float32),
            pltpu.VMEM((_CH,), jnp.int32),
            pltpu.VMEM((_CH,), jnp.float32),
            pltpu.VMEM_SHARED((_NP,), jnp.float32),
        ],
        compiler_params=pltpu.CompilerParams(needs_layout_passes=False),
    )


def _tc_combine(p0):
    def body(p_ref, o_ref):
        o_ref[...] = jnp.sum(p_ref[...], axis=1)
    return pl.pallas_call(
        body, out_shape=jax.ShapeDtypeStruct((2, _NP), jnp.float32))(p0)


def _tc_dense0(a0rs, W, b):
    def body(a_ref, W_ref, b_ref, o_ref):
        o_ref[...] = W_ref[0:1, 0:1] * a_ref[0:1] + b_ref[...] * a_ref[1:2]
    return pl.pallas_call(
        body, out_shape=jax.ShapeDtypeStruct((1, _NP), jnp.float32))(
            a0rs, W, b.reshape(1, 1))


def _tc_dense(p, a0rs, W, b):
    d = W.shape[0]
    def body(p_ref, a_ref, W_ref, b_ref, o_ref):
        u = jnp.sum(p_ref[...], axis=1)
        u = jnp.concatenate([u, a_ref[0:1]], axis=0)
        s = lax.dot_general(W_ref[...], u, (((1,), (0,)), ((), ())),
                            precision=lax.Precision.HIGHEST,
                            preferred_element_type=jnp.float32)
        o_ref[...] = s + b_ref[...] * a_ref[1:2]
    return pl.pallas_call(
        body, out_shape=jax.ShapeDtypeStruct((d, _NP), jnp.float32))(
            p, a0rs, W, b.reshape(d, 1))


def _tc_dnn(hp, W2, b2):
    m, dp = hp.shape
    def body(h_ref, W_ref, b_ref, o_ref):
        o_ref[...] = lax.dot_general(
            h_ref[...], W_ref[...], (((1,), (1,)), ((), ())),
            precision=lax.Precision.HIGHEST,
            preferred_element_type=jnp.float32) + b_ref[...]
    return pl.pallas_call(
        body, out_shape=jax.ShapeDtypeStruct((m, dp), jnp.float32))(
            hp, W2, b2.reshape(1, dp))


def kernel(h_0, edge_index, edge_values, gnn_W, gnn_b, dnn_W, dnn_b):
    row = edge_index[0].astype(jnp.int32)
    col = edge_index[1].astype(jnp.int32)
    rc = jnp.bitwise_or(jnp.left_shift(row, 16), col)
    val = edge_values.astype(jnp.float32)

    pad = _NP - _N
    h0c = jnp.pad(h_0[:, 0], (0, pad))
    onesc = jnp.pad(jnp.ones((_N,), jnp.float32), (0, pad))
    init_cols = jnp.stack([h0c, onesc])

    p0 = _make_sc_spmm(2)(init_cols.reshape(-1), rc, val)
    a0rs = _tc_combine(p0.reshape(2, _NW, _NP))

    g = None
    for i in range(_NUM_LAYER):
        d = i + 1
        if i == 0:
            s = _tc_dense0(a0rs, gnn_W[0], gnn_b[0])
        else:
            p = _make_sc_spmm(i)(g.reshape(-1), rc, val)
            s = _tc_dense(p.reshape(i, _NW, _NP), a0rs,
                          gnn_W[i], gnn_b[i])
        tail = s[:, _N - _P:_N]
        hp = tail.T.reshape(_P // 5, 5 * d)
        hp2 = _tc_dnn(hp, dnn_W[i], dnn_b[i])
        tail2 = hp2.reshape(_P, d).T
        g = jnp.concatenate(
            [s[:, :_N - _P], tail2,
             jnp.zeros((d, pad), jnp.float32)], axis=1)

    out = jnp.concatenate([g[:, _N - _P:_N].T, h_0[-_P:]], axis=1)
    return out


# speedup vs baseline: 2.4466x; 2.4466x over previous
import functools

import jax
import jax.numpy as jnp
from jax import lax
from jax.experimental import pallas as pl
from jax.experimental.pallas import tpu as pltpu
from jax.experimental.pallas import tpu_sc as plsc

_N = 50000
_E = 1600000
_P = 25000
_NUM_LAYER = 6

_NC = 2
_NS = 16
_NW = _NC * _NS
_STRIPE = 3136
_NP = _NS * _STRIPE
_EPW = _E // _NW
_CH = 2000
_NCHUNK = _EPW // _CH


def _sc_spmm_body(w, cols_hbm, rc_hbm, val_hbm, out_hbm,
                  htab, acc, rcb, valb, bcast):
    cid = lax.axis_index("c")
    sid = lax.axis_index("s")
    wid = cid * _NS + sid
    ebase = wid * _EPW
    zeros16 = jnp.zeros((16,), jnp.float32)

    for c in range(w):
        @pl.when(sid == 0)
        def _():
            pltpu.sync_copy(cols_hbm.at[pl.ds(c * _NP, _NP)], bcast)
        plsc.subcore_barrier()
        pltpu.sync_copy(bcast, htab)

        @plsc.parallel_loop(0, _NP // 16, 1, unroll=8)
        def _(j):
            acc[pl.ds(j * 16, 16)] = zeros16

        def chunk_body(ck, carry):
            base = ebase + ck * _CH
            pltpu.sync_copy(rc_hbm.at[pl.ds(base, _CH)], rcb)
            pltpu.sync_copy(val_hbm.at[pl.ds(base, _CH)], valb)

            @plsc.parallel_loop(0, _CH // 16, 1, unroll=8)
            def _(t):
                rc = rcb[pl.ds(t * 16, 16)]
                vv = valb[pl.ds(t * 16, 16)]
                colv = lax.bitwise_and(rc, jnp.int32(0xFFFF))
                rowv = lax.shift_right_logical(rc, 16)
                g = plsc.load_gather(htab, [colv])
                plsc.addupdate_scatter(acc, [rowv], vv * g)
            return carry
        lax.fori_loop(0, _NCHUNK, chunk_body, 0)

        pltpu.sync_copy(acc, out_hbm.at[pl.ds((c * _NW + wid) * _NP, _NP)])
        plsc.subcore_barrier()


@functools.cache
def _make_sc_spmm(w):
    mesh = plsc.VectorSubcoreMesh(core_axis_name="c", subcore_axis_name="s", num_cores=_NC, num_subcores=_NS)
    return pl.kernel(
        functools.partial(_sc_spmm_body, w),
        out_type=jax.ShapeDtypeStruct((w * _NW * _NP,), jnp.float32),
        mesh=mesh,
        scratch_types=[
            pltpu.VMEM((_NP,), jnp.float32),
            pltpu.VMEM((_NP,), jnp.float32),
            pltpu.VMEM((_CH,), jnp.int32),
            pltpu.VMEM((_CH,), jnp.float32),
            pltpu.VMEM_SHARED((_NP,), jnp.float32),
        ],
        compiler_params=pltpu.CompilerParams(needs_layout_passes=False),
    )


def _tc_combine(p0):
    def body(p_ref, o_ref):
        o_ref[...] = jnp.sum(p_ref[...], axis=1)
    return pl.pallas_call(
        body, out_shape=jax.ShapeDtypeStruct((2, _NP), jnp.float32))(p0)


def _tc_dense0(a0rs, W, b):
    def body(a_ref, W_ref, b_ref, o_ref):
        o_ref[...] = W_ref[0:1, 0:1] * a_ref[0:1] + b_ref[...] * a_ref[1:2]
    return pl.pallas_call(
        body, out_shape=jax.ShapeDtypeStruct((1, _NP), jnp.float32))(
            a0rs, W, b.reshape(1, 1))


def _tc_dense(p, a0rs, W, b):
    d = W.shape[0]
    def body(p_ref, a_ref, W_ref, b_ref, o_ref):
        u = jnp.sum(p_ref[...], axis=1)
        u = jnp.concatenate([u, a_ref[0:1]], axis=0)
        s = lax.dot_general(W_ref[...], u, (((1,), (0,)), ((), ())),
                            precision=lax.Precision.HIGHEST,
                            preferred_element_type=jnp.float32)
        o_ref[...] = s + b_ref[...] * a_ref[1:2]
    return pl.pallas_call(
        body, out_shape=jax.ShapeDtypeStruct((d, _NP), jnp.float32))(
            p, a0rs, W, b.reshape(d, 1))


def _tc_dnn(hp, W2, b2):
    m, dp = hp.shape
    def body(h_ref, W_ref, b_ref, o_ref):
        o_ref[...] = lax.dot_general(
            h_ref[...], W_ref[...], (((1,), (1,)), ((), ())),
            precision=lax.Precision.HIGHEST,
            preferred_element_type=jnp.float32) + b_ref[...]
    return pl.pallas_call(
        body, out_shape=jax.ShapeDtypeStruct((m, dp), jnp.float32))(
            hp, W2, b2.reshape(1, dp))


def kernel(h_0, edge_index, edge_values, gnn_W, gnn_b, dnn_W, dnn_b):
    row = edge_index[0].astype(jnp.int32)
    col = edge_index[1].astype(jnp.int32)
    rc = jnp.bitwise_or(jnp.left_shift(row, 16), col)
    val = edge_values.astype(jnp.float32)

    pad = _NP - _N
    h0c = jnp.pad(h_0[:, 0], (0, pad))
    onesc = jnp.pad(jnp.ones((_N,), jnp.float32), (0, pad))
    init_cols = jnp.stack([h0c, onesc])

    p0 = _make_sc_spmm(2)(init_cols.reshape(-1), rc, val)
    a0rs = _tc_combine(p0.reshape(2, _NW, _NP))

    g = None
    for i in range(_NUM_LAYER):
        d = i + 1
        if i == 0:
            s = _tc_dense0(a0rs, gnn_W[0], gnn_b[0])
        else:
            p = _make_sc_spmm(i)(g.reshape(-1), rc, val)
            s = _tc_dense(p.reshape(i, _NW, _NP), a0rs,
                          gnn_W[i], gnn_b[i])
        tail = s[:, _N - _P:_N]
        hp = tail.T.reshape(_P // 5, 5 * d)
        hp2 = _tc_dnn(hp, dnn_W[i], dnn_b[i])
        tail2 = hp2.reshape(_P, d).T
        g = jnp.concatenate(
            [s[:, :_N - _P], tail2,
             jnp.zeros((d, pad), jnp.float32)], axis=1)

    out = jnp.concatenate([g[:, _N - _P:_N].T, h_0[-_P:]], axis=1)
    return out


# speedup vs baseline: 3.5356x; 1.4451x over previous
import functools

import jax
import jax.numpy as jnp
from jax import lax
from jax.experimental import pallas as pl
from jax.experimental.pallas import tpu as pltpu
from jax.experimental.pallas import tpu_sc as plsc

_N = 50000
_E = 1600000
_P = 25000
_NUM_LAYER = 6

_NC = 2
_NS = 16
_NW = _NC * _NS
_STRIPE = 3136
_NP = _NS * _STRIPE
_EPW = _E // _NW
_CH = 2000
_NCHUNK = _EPW // _CH


def _sc_spmm_body(w, cols_hbm, rc_hbm, val_hbm, out_hbm,
                  htab, acc, rcb0, valb0, rcb1, valb1, bcast,
                  sr0, sv0, sr1, sv1):
    cid = lax.axis_index("c")
    sid = lax.axis_index("s")
    wid = cid * _NS + sid
    ebase = wid * _EPW
    zeros16 = jnp.zeros((16,), jnp.float32)

    def _issue(ck, rcb, valb, sr, sv):
        base = ebase + ck * _CH
        pltpu.async_copy(rc_hbm.at[pl.ds(base, _CH)], rcb, sr)
        pltpu.async_copy(val_hbm.at[pl.ds(base, _CH)], valb, sv)

    def _wait(ck, rcb, valb, sr, sv):
        base = ebase + ck * _CH
        pltpu.make_async_copy(rc_hbm.at[pl.ds(base, _CH)], rcb, sr).wait()
        pltpu.make_async_copy(val_hbm.at[pl.ds(base, _CH)], valb, sv).wait()

    def _process(rcb, valb):
        @plsc.parallel_loop(0, _CH // 16, 1, unroll=8)
        def _(t):
            rc = rcb[pl.ds(t * 16, 16)]
            vv = valb[pl.ds(t * 16, 16)]
            colv = lax.bitwise_and(rc, jnp.int32(0xFFFF))
            rowv = lax.shift_right_logical(rc, 16)
            g = plsc.load_gather(htab, [colv])
            plsc.addupdate_scatter(acc, [rowv], vv * g)

    for c in range(w):
        @pl.when(sid == 0)
        def _():
            pltpu.sync_copy(cols_hbm.at[pl.ds(c * _NP, _NP)], bcast)
        plsc.subcore_barrier()
        pltpu.sync_copy(bcast, htab)

        @plsc.parallel_loop(0, _NP // 16, 1, unroll=8)
        def _(j):
            acc[pl.ds(j * 16, 16)] = zeros16

        _issue(0, rcb0, valb0, sr0, sv0)

        def chunk_body(ck, carry):
            @pl.when(ck % 2 == 0)
            def _():
                _wait(ck, rcb0, valb0, sr0, sv0)

                @pl.when(ck + 1 < _NCHUNK)
                def _():
                    _issue(ck + 1, rcb1, valb1, sr1, sv1)
                _process(rcb0, valb0)

            @pl.when(ck % 2 == 1)
            def _():
                _wait(ck, rcb1, valb1, sr1, sv1)

                @pl.when(ck + 1 < _NCHUNK)
                def _():
                    _issue(ck + 1, rcb0, valb0, sr0, sv0)
                _process(rcb1, valb1)
            return carry
        lax.fori_loop(0, _NCHUNK, chunk_body, 0)

        pltpu.sync_copy(acc, out_hbm.at[pl.ds((c * _NW + wid) * _NP, _NP)])
        plsc.subcore_barrier()


@functools.cache
def _make_sc_spmm(w):
    mesh = plsc.VectorSubcoreMesh(core_axis_name="c", subcore_axis_name="s", num_cores=_NC, num_subcores=_NS)
    return pl.kernel(
        functools.partial(_sc_spmm_body, w),
        out_type=jax.ShapeDtypeStruct((w * _NW * _NP,), jnp.float32),
        mesh=mesh,
        scratch_types=[
            pltpu.VMEM((_NP,), jnp.float32),
            pltpu.VMEM((_NP,), jnp.float32),
            pltpu.VMEM((_CH,), jnp.int32),
            pltpu.VMEM((_CH,), jnp.float32),
            pltpu.VMEM((_CH,), jnp.int32),
            pltpu.VMEM((_CH,), jnp.float32),
            pltpu.VMEM_SHARED((_NP,), jnp.float32),
            pltpu.SemaphoreType.DMA,
            pltpu.SemaphoreType.DMA,
            pltpu.SemaphoreType.DMA,
            pltpu.SemaphoreType.DMA,
        ],
        compiler_params=pltpu.CompilerParams(needs_layout_passes=False),
    )


def _tc_combine(p0):
    def body(p_ref, o_ref):
        o_ref[...] = jnp.sum(p_ref[...], axis=1)
    return pl.pallas_call(
        body, out_shape=jax.ShapeDtypeStruct((2, _NP), jnp.float32))(p0)


def _tc_dense0(a0rs, W, b):
    def body(a_ref, W_ref, b_ref, o_ref):
        o_ref[...] = W_ref[0:1, 0:1] * a_ref[0:1] + b_ref[...] * a_ref[1:2]
    return pl.pallas_call(
        body, out_shape=jax.ShapeDtypeStruct((1, _NP), jnp.float32))(
            a0rs, W, b.reshape(1, 1))


def _tc_dense(p, a0rs, W, b):
    d = W.shape[0]
    def body(p_ref, a_ref, W_ref, b_ref, o_ref):
        u = jnp.sum(p_ref[...], axis=1)
        u = jnp.concatenate([u, a_ref[0:1]], axis=0)
        s = lax.dot_general(W_ref[...], u, (((1,), (0,)), ((), ())),
                            precision=lax.Precision.HIGHEST,
                            preferred_element_type=jnp.float32)
        o_ref[...] = s + b_ref[...] * a_ref[1:2]
    return pl.pallas_call(
        body, out_shape=jax.ShapeDtypeStruct((d, _NP), jnp.float32))(
            p, a0rs, W, b.reshape(d, 1))


def _tc_dnn(hp, W2, b2):
    m, dp = hp.shape
    def body(h_ref, W_ref, b_ref, o_ref):
        o_ref[...] = lax.dot_general(
            h_ref[...], W_ref[...], (((1,), (1,)), ((), ())),
            precision=lax.Precision.HIGHEST,
            preferred_element_type=jnp.float32) + b_ref[...]
    return pl.pallas_call(
        body, out_shape=jax.ShapeDtypeStruct((m, dp), jnp.float32))(
            hp, W2, b2.reshape(1, dp))


def kernel(h_0, edge_index, edge_values, gnn_W, gnn_b, dnn_W, dnn_b):
    row = edge_index[0].astype(jnp.int32)
    col = edge_index[1].astype(jnp.int32)
    rc = jnp.bitwise_or(jnp.left_shift(row, 16), col)
    val = edge_values.astype(jnp.float32)

    pad = _NP - _N
    h0c = jnp.pad(h_0[:, 0], (0, pad))
    onesc = jnp.pad(jnp.ones((_N,), jnp.float32), (0, pad))
    init_cols = jnp.stack([h0c, onesc])

    p0 = _make_sc_spmm(2)(init_cols.reshape(-1), rc, val)
    a0rs = _tc_combine(p0.reshape(2, _NW, _NP))

    g = None
    for i in range(_NUM_LAYER):
        d = i + 1
        if i == 0:
            s = _tc_dense0(a0rs, gnn_W[0], gnn_b[0])
        else:
            p = _make_sc_spmm(i)(g.reshape(-1), rc, val)
            s = _tc_dense(p.reshape(i, _NW, _NP), a0rs,
                          gnn_W[i], gnn_b[i])
        tail = s[:, _N - _P:_N]
        hp = tail.T.reshape(_P // 5, 5 * d)
        hp2 = _tc_dnn(hp, dnn_W[i], dnn_b[i])
        tail2 = hp2.reshape(_P, d).T
        g = jnp.concatenate(
            [s[:, :_N - _P], tail2,
             jnp.zeros((d, pad), jnp.float32)], axis=1)

    out = jnp.concatenate([g[:, _N - _P:_N].T, h_0[-_P:]], axis=1)
    return out
